# baseline (speedup 1.0000x reference)
import functools
import math

import jax
import jax.numpy as jnp
from jax import lax
from jax.experimental import pallas as pl
from jax.experimental.pallas import tpu as pltpu
from jax.experimental.pallas import tpu_sc as plsc

_BATCH = 16384
_SEQ = 20
_TABLE = 1000
_TABLE_PAD = 1008
_NC = 2
_NS = 16
_NW = _NC * _NS
_ROWS_PER_W = _BATCH // _NW
_IDX_PER_W = _ROWS_PER_W * _SEQ
_GROUPS = _ROWS_PER_W // 16

_LEVEL_SCALE = math.exp(math.log(1e50) / 20.0) ** 10.0


def _sc_body(idx_hbm, table_hbm, out_hbm, idx_v, table_v, out_v, red_v,
             sem0, semt):
    wid = lax.axis_index("s") * _NC + lax.axis_index("c")

    idx_dma = pltpu.async_copy(idx_hbm.at[wid], idx_v, sem0)
    tab_dma = pltpu.async_copy(table_hbm, table_v.at[pl.ds(0, _TABLE)], semt)
    tab_dma.wait()
    lane = jnp.arange(16, dtype=jnp.int32)
    tail = table_v[pl.ds(_TABLE_PAD - 16, 16)]
    table_v[pl.ds(_TABLE_PAD - 16, 16)] = jnp.where(
        lane < 16 - (_TABLE_PAD - _TABLE), tail, jnp.float32(0.0)
    )

    def ssq_step(k, a):
        v = table_v[pl.ds(k * 16, 16)]
        return a + v * v

    ssq = lax.fori_loop(
        0, _TABLE_PAD // 16, ssq_step, jnp.zeros((16,), jnp.float32)
    )
    for off in (8, 4, 2, 1):
        red_v[...] = ssq
        ssq = ssq + plsc.load_gather(red_v, [(lane + off) & 15])
    s = ssq
    s2 = s * s
    s4 = s2 * s2
    s10 = s4 * s4 * s2
    inv_denom = jnp.float32(1.0) / s10
    lvl = jnp.float32(_LEVEL_SCALE)

    def group(g, _):
        base = g * 16
        ones = jnp.ones((16,), jnp.float32)

        def jstep(j, accs):
            a0, a1 = accs
            inds = idx_v[j, pl.ds(base, 16)]
            vals = plsc.load_gather(table_v, [inds])
            return (a1, a0 * vals)

        a0, a1 = lax.fori_loop(0, _SEQ, jstep, (ones, ones), unroll=4)
        acc = a0 * a1 * inv_denom * lvl
        out_v[pl.ds(base, 16)] = acc
        return _

    idx_dma.wait()
    lax.fori_loop(0, _GROUPS, group, 0)

    pltpu.sync_copy(out_v, out_hbm.at[pl.ds(wid * _ROWS_PER_W, _ROWS_PER_W)])


@functools.partial(jax.jit)
def _run(idx_flat, table_pad):
    call = pl.kernel(
        _sc_body,
        out_type=jax.ShapeDtypeStruct((_BATCH,), jnp.float32),
        mesh=plsc.VectorSubcoreMesh(core_axis_name="c", subcore_axis_name="s"),
        scratch_types=[
            pltpu.VMEM((_SEQ, _ROWS_PER_W), jnp.int32),
            pltpu.VMEM((_TABLE_PAD,), jnp.float32),
            pltpu.VMEM((_ROWS_PER_W,), jnp.float32),
            pltpu.VMEM((16,), jnp.float32),
            pltpu.SemaphoreType.DMA,
            pltpu.SemaphoreType.DMA,
        ],
        compiler_params=pltpu.CompilerParams(needs_layout_passes=False),
    )
    return call(idx_flat, table_pad)


def kernel(_input, mat0):
    idx3 = _input.reshape(_NW, _ROWS_PER_W, _SEQ).transpose(0, 2, 1)
    return _run(idx3, mat0)


# speedup vs baseline: 1.0038x; 1.0038x over previous
import functools
import math

import jax
import jax.numpy as jnp
from jax import lax
from jax.experimental import pallas as pl
from jax.experimental.pallas import tpu as pltpu
from jax.experimental.pallas import tpu_sc as plsc

_BATCH = 16384
_SEQ = 20
_TABLE = 1000
_TABLE_PAD = 1008
_NC = 2
_NS = 16
_NW = _NC * _NS
_ROWS_PER_W = _BATCH // _NW
_GROUPS = _ROWS_PER_W // 16

_LEVEL_SCALE = math.exp(math.log(1e50) / 20.0) ** 10.0


def _sc_body(idx_hbm, table_hbm, out_hbm, idx_v, table_v, out_v, red_v,
             sem0, semt):
    wid = lax.axis_index("s") * _NC + lax.axis_index("c")

    idx_dma = pltpu.async_copy(idx_hbm.at[wid], idx_v, sem0)
    tab_dma = pltpu.async_copy(table_hbm, table_v.at[pl.ds(0, _TABLE)], semt)
    tab_dma.wait()
    lane = jnp.arange(16, dtype=jnp.int32)
    tail = table_v[pl.ds(_TABLE_PAD - 16, 16)]
    table_v[pl.ds(_TABLE_PAD - 16, 16)] = jnp.where(
        lane < 16 - (_TABLE_PAD - _TABLE), tail, jnp.float32(0.0)
    )

    def ssq_step(k, a):
        v = table_v[pl.ds(k * 16, 16)]
        return a + v * v

    ssq = lax.fori_loop(
        0, _TABLE_PAD // 16, ssq_step, jnp.zeros((16,), jnp.float32)
    )
    for off in (8, 4, 2, 1):
        red_v[...] = ssq
        ssq = ssq + plsc.load_gather(red_v, [(lane + off) & 15])
    s = ssq
    s2 = s * s
    s4 = s2 * s2
    s10 = s4 * s4 * s2
    inv_denom = jnp.float32(1.0) / s10
    lvl = jnp.float32(_LEVEL_SCALE)

    def group(g, _):
        base = g * 16
        ones = jnp.ones((16,), jnp.float32)

        def jstep(j, accs):
            a0, a1 = accs
            inds = idx_v[j, pl.ds(base, 16)]
            vals = plsc.load_gather(table_v, [inds])
            return (a1, a0 * vals)

        a0, a1 = lax.fori_loop(0, _SEQ, jstep, (ones, ones), unroll=4)
        acc = a0 * a1 * inv_denom * lvl
        out_v[pl.ds(base, 16)] = acc
        return _

    idx_dma.wait()
    lax.fori_loop(0, _GROUPS, group, 0)

    pltpu.sync_copy(out_v, out_hbm.at[pl.ds(wid * _ROWS_PER_W, _ROWS_PER_W)])


@functools.partial(jax.jit)
def _run(idx3, table):
    call = pl.kernel(
        _sc_body,
        out_type=jax.ShapeDtypeStruct((_BATCH,), jnp.float32),
        mesh=plsc.VectorSubcoreMesh(core_axis_name="c", subcore_axis_name="s"),
        scratch_types=[
            pltpu.VMEM((_SEQ, _ROWS_PER_W), jnp.int32),
            pltpu.VMEM((_TABLE_PAD,), jnp.float32),
            pltpu.VMEM((_ROWS_PER_W,), jnp.float32),
            pltpu.VMEM((16,), jnp.float32),
            pltpu.SemaphoreType.DMA,
            pltpu.SemaphoreType.DMA,
        ],
        compiler_params=pltpu.CompilerParams(needs_layout_passes=False),
    )
    return call(idx3, table)


def kernel(_input, mat0):
    idx3 = _input.reshape(_NW, _ROWS_PER_W, _SEQ).transpose(0, 2, 1)
    return _run(idx3, mat0)
